# baseline (speedup 1.0000x reference)
import functools

import jax
import jax.numpy as jnp
from jax import lax
from jax.experimental import pallas as pl
from jax.experimental.pallas import tpu as pltpu
from jax.experimental.pallas import tpu_sc as plsc

_NC = 2
_NS = 16
_NW = _NC * _NS
_CHUNK = 1024
_NBUF = 2
_LAG = 1
_HP = 32


def _sc_mesh():
    return plsc.VectorSubcoreMesh(core_axis_name="c", subcore_axis_name="s")


def _degrees(dst3, ones_l, zeros_n, np_, nch):
    rpt = np_ // _NS

    @functools.partial(
        pl.kernel,
        out_type=[jax.ShapeDtypeStruct((np_,), jnp.float32),
                  jax.ShapeDtypeStruct((np_,), jnp.float32)],
        mesh=_sc_mesh(),
        scratch_types=[
            pltpu.VMEM((nch, _CHUNK), jnp.int32),
            pltpu.VMEM((_CHUNK,), jnp.float32),
            pltpu.VMEM_SHARED((np_,), jnp.float32),
            pltpu.SemaphoreType.DMA,
        ],
        compiler_params=pltpu.CompilerParams(use_tc_tiling_on_sc=False),
    )
    def k(dst_hbm, ones_hbm, z_hbm, out0, out1, dst_v, ones_v, acc_sh, sem):
        c = lax.axis_index("c")
        s = lax.axis_index("s")
        wid = c * _NS + s
        base = s * rpt
        pltpu.sync_copy(z_hbm.at[pl.ds(base, rpt)], acc_sh.at[pl.ds(base, rpt)])
        pltpu.sync_copy(dst_hbm.at[wid], dst_v)
        pltpu.sync_copy(ones_hbm, ones_v)
        plsc.subcore_barrier()

        def body(j, carry):
            pltpu.async_copy(ones_v, acc_sh.at[dst_v.at[j]], sem, add=True)
            return carry

        lax.fori_loop(0, nch, body, 0)

        def drain(j, carry):
            pltpu.make_async_copy(ones_v, acc_sh.at[dst_v.at[0]], sem).wait()
            return carry

        lax.fori_loop(0, nch, drain, 0)
        plsc.subcore_barrier()

        @pl.when(c == 0)
        def _():
            pltpu.sync_copy(acc_sh.at[pl.ds(base, rpt)], out0.at[pl.ds(base, rpt)])

        @pl.when(c == 1)
        def _():
            pltpu.sync_copy(acc_sh.at[pl.ds(base, rpt)], out1.at[pl.ds(base, rpt)])

    return k(dst3, ones_l, zeros_n)


def _scatter_rows(src3, dst3, g, zeros_rows, np_, nch):
    rpt = np_ // _NS

    @functools.partial(
        pl.kernel,
        out_type=[jax.ShapeDtypeStruct((np_, _HP), jnp.float32),
                  jax.ShapeDtypeStruct((np_, _HP), jnp.float32)],
        mesh=_sc_mesh(),
        scratch_types=[
            pltpu.VMEM((nch, _CHUNK), jnp.int32),
            pltpu.VMEM((nch, _CHUNK), jnp.int32),
            pltpu.VMEM((_NBUF, _CHUNK, _HP), jnp.float32),
            pltpu.VMEM_SHARED((np_, _HP), jnp.float32),
            pltpu.SemaphoreType.DMA((_NBUF,)),
            pltpu.SemaphoreType.DMA((_NBUF,)),
        ],
        compiler_params=pltpu.CompilerParams(use_tc_tiling_on_sc=False),
    )
    def k(src_hbm, dst_hbm, g_hbm, z_hbm, out0, out1,
          src_v, dst_v, rows_v, acc_sh, gsem, ssem):
        c = lax.axis_index("c")
        s = lax.axis_index("s")
        wid = c * _NS + s
        base = s * rpt

        @pl.when(c == 0)
        def _():
            pltpu.sync_copy(g_hbm.at[pl.ds(base, rpt)], acc_sh.at[pl.ds(base, rpt)])

        @pl.when(c == 1)
        def _():
            pltpu.sync_copy(z_hbm.at[pl.ds(base, rpt)], acc_sh.at[pl.ds(base, rpt)])

        pltpu.sync_copy(src_hbm.at[wid], src_v)
        pltpu.sync_copy(dst_hbm.at[wid], dst_v)
        plsc.subcore_barrier()

        def body(j, carry):
            b = lax.rem(j, _NBUF)

            @pl.when(j < nch)
            def _():
                @pl.when(j >= _NBUF)
                def _():
                    pltpu.make_async_copy(
                        rows_v.at[b], acc_sh.at[dst_v.at[j - _NBUF]],
                        ssem.at[b]).wait()
                pltpu.async_copy(g_hbm.at[src_v.at[j]], rows_v.at[b],
                                 gsem.at[b])

            i = j - _LAG

            @pl.when((i >= 0) & (i < nch))
            def _():
                bi = lax.rem(i + _NBUF, _NBUF)
                pltpu.make_async_copy(g_hbm.at[src_v.at[i]], rows_v.at[bi],
                                      gsem.at[bi]).wait()
                pltpu.async_copy(rows_v.at[bi], acc_sh.at[dst_v.at[i]],
                                 ssem.at[bi], add=True)

            return carry

        lax.fori_loop(0, nch + _LAG, body, 0)
        for b in range(_NBUF):
            pltpu.make_async_copy(rows_v.at[b], acc_sh.at[dst_v.at[0]],
                                  ssem.at[b]).wait()
        plsc.subcore_barrier()

        @pl.when(c == 0)
        def _():
            pltpu.sync_copy(acc_sh.at[pl.ds(base, rpt)], out0.at[pl.ds(base, rpt)])

        @pl.when(c == 1)
        def _():
            pltpu.sync_copy(acc_sh.at[pl.ds(base, rpt)], out1.at[pl.ds(base, rpt)])

    return k(src3, dst3, g, zeros_rows)


_BN = 2048


def _tc_matmul0(xp, w0p):
    np_, f_in = xp.shape

    def body(x_ref, w_ref, h_ref):
        h_ref[...] = jnp.dot(x_ref[...], w_ref[...],
                             preferred_element_type=jnp.float32)

    return pl.pallas_call(
        body,
        grid=(np_ // _BN,),
        in_specs=[
            pl.BlockSpec((_BN, f_in), lambda i: (i, 0)),
            pl.BlockSpec((f_in, _HP), lambda i: (0, 0)),
        ],
        out_specs=pl.BlockSpec((_BN, _HP), lambda i: (i, 0)),
        out_shape=jax.ShapeDtypeStruct((np_, _HP), jnp.float32),
    )(xp, w0p)


def _tc_prep(h0, d0, d1):
    np_ = h0.shape[0]

    def body(h_ref, d0_ref, d1_ref, g_ref, dinv_ref):
        dinv = lax.rsqrt(d0_ref[...] + d1_ref[...] + 1.0)
        g_ref[...] = h_ref[...] * dinv
        dinv_ref[...] = dinv

    return pl.pallas_call(
        body,
        grid=(np_ // _BN,),
        in_specs=[
            pl.BlockSpec((_BN, _HP), lambda i: (i, 0)),
            pl.BlockSpec((_BN, 1), lambda i: (i, 0)),
            pl.BlockSpec((_BN, 1), lambda i: (i, 0)),
        ],
        out_specs=[
            pl.BlockSpec((_BN, _HP), lambda i: (i, 0)),
            pl.BlockSpec((_BN, 1), lambda i: (i, 0)),
        ],
        out_shape=[
            jax.ShapeDtypeStruct((np_, _HP), jnp.float32),
            jax.ShapeDtypeStruct((np_, 1), jnp.float32),
        ],
    )(h0, d0, d1)


def _tc_mid(p0, p1, dinv, b_prev, w_next):
    np_ = p0.shape[0]

    def body(p0_ref, p1_ref, dinv_ref, b_ref, w_ref, g_ref):
        dinv = dinv_ref[...]
        h = jnp.maximum((p0_ref[...] + p1_ref[...]) * dinv + b_ref[...], 0.0)
        g_ref[...] = jnp.dot(h, w_ref[...],
                             preferred_element_type=jnp.float32) * dinv

    return pl.pallas_call(
        body,
        grid=(np_ // _BN,),
        in_specs=[
            pl.BlockSpec((_BN, _HP), lambda i: (i, 0)),
            pl.BlockSpec((_BN, _HP), lambda i: (i, 0)),
            pl.BlockSpec((_BN, 1), lambda i: (i, 0)),
            pl.BlockSpec((1, _HP), lambda i: (0, 0)),
            pl.BlockSpec((_HP, _HP), lambda i: (0, 0)),
        ],
        out_specs=pl.BlockSpec((_BN, _HP), lambda i: (i, 0)),
        out_shape=jax.ShapeDtypeStruct((np_, _HP), jnp.float32),
    )(p0, p1, dinv, b_prev, w_next)


def _tc_final(p0, p1, dinv, b3p, wlp, blp):
    np_ = p0.shape[0]
    n_cls = wlp.shape[1]

    def body(p0_ref, p1_ref, dinv_ref, b_ref, w_ref, bl_ref, o_ref):
        h = (p0_ref[...] + p1_ref[...]) * dinv_ref[...] + b_ref[...]
        o_ref[...] = jnp.dot(h, w_ref[...],
                             preferred_element_type=jnp.float32) + bl_ref[...]

    return pl.pallas_call(
        body,
        grid=(np_ // _BN,),
        in_specs=[
            pl.BlockSpec((_BN, _HP), lambda i: (i, 0)),
            pl.BlockSpec((_BN, _HP), lambda i: (i, 0)),
            pl.BlockSpec((_BN, 1), lambda i: (i, 0)),
            pl.BlockSpec((1, _HP), lambda i: (0, 0)),
            pl.BlockSpec((_HP, n_cls), lambda i: (0, 0)),
            pl.BlockSpec((1, n_cls), lambda i: (0, 0)),
        ],
        out_specs=pl.BlockSpec((_BN, n_cls), lambda i: (i, 0)),
        out_shape=jax.ShapeDtypeStruct((np_, n_cls), jnp.float32),
    )(p0, p1, dinv, b3p, wlp, blp)


def kernel(x, edge_index, batch, W0, b0, W1, b1, W2, b2, W3, b3, Wl, bl):
    n, f_in = x.shape
    e = edge_index.shape[1]
    h = W0.shape[1]

    np_ = ((n + _BN - 1) // _BN) * _BN
    n_pad_rows = np_ - n

    slab = _NW * _CHUNK
    ep = ((e + slab - 1) // slab) * slab
    nch = ep // slab
    pad = ep - e
    pad_idx = n + (jnp.arange(pad, dtype=jnp.int32) % n_pad_rows)
    src3 = jnp.concatenate([edge_index[0], pad_idx]).reshape(_NW, nch, _CHUNK)
    dst3 = jnp.concatenate([edge_index[1], pad_idx]).reshape(_NW, nch, _CHUNK)

    zeros_rows = jnp.zeros((np_, _HP), jnp.float32)
    zeros_n = jnp.zeros((np_,), jnp.float32)
    ones_l = jnp.ones((_CHUNK,), jnp.float32)

    xp = jnp.pad(x, ((0, np_ - n), (0, 0)))
    w0p = jnp.pad(W0, ((0, 0), (0, _HP - h)))
    w1p = jnp.pad(W1, ((0, _HP - h), (0, _HP - h)))
    w2p = jnp.pad(W2, ((0, _HP - h), (0, _HP - h)))
    w3p = jnp.pad(W3, ((0, _HP - h), (0, _HP - h)))
    wlp = jnp.pad(Wl, ((0, _HP - h), (0, 0)))
    b0p = jnp.pad(b0, (0, _HP - h))[None, :]
    b1p = jnp.pad(b1, (0, _HP - h))[None, :]
    b2p = jnp.pad(b2, (0, _HP - h))[None, :]
    b3p = jnp.pad(b3, (0, _HP - h))[None, :]
    blp = bl[None, :]

    h0 = _tc_matmul0(xp, w0p)
    d0, d1 = _degrees(dst3, ones_l, zeros_n, np_, nch)
    g, dinv = _tc_prep(h0, d0[:, None], d1[:, None])

    for b_prev, w_next in ((b0p, w1p), (b1p, w2p), (b2p, w3p)):
        p0, p1 = _scatter_rows(src3, dst3, g, zeros_rows, np_, nch)
        g = _tc_mid(p0, p1, dinv, b_prev, w_next)

    p0, p1 = _scatter_rows(src3, dst3, g, zeros_rows, np_, nch)
    out = _tc_final(p0, p1, dinv, b3p, wlp, blp)
    return out[:n]


# speedup vs baseline: 1.0888x; 1.0888x over previous
import functools

import jax
import jax.numpy as jnp
from jax import lax
from jax.experimental import pallas as pl
from jax.experimental.pallas import tpu as pltpu
from jax.experimental.pallas import tpu_sc as plsc

_NC = 2
_NS = 16
_NW = _NC * _NS
_CHUNK = 512
_NBUF = 4
_LAG = 2
_HP = 32


def _sc_mesh():
    return plsc.VectorSubcoreMesh(core_axis_name="c", subcore_axis_name="s")


def _degrees(dst3, ones_l, zeros_n, np_, nch):
    rpt = np_ // _NS

    @functools.partial(
        pl.kernel,
        out_type=[jax.ShapeDtypeStruct((np_,), jnp.float32),
                  jax.ShapeDtypeStruct((np_,), jnp.float32)],
        mesh=_sc_mesh(),
        scratch_types=[
            pltpu.VMEM((nch, _CHUNK), jnp.int32),
            pltpu.VMEM((_CHUNK,), jnp.float32),
            pltpu.VMEM_SHARED((np_,), jnp.float32),
            pltpu.SemaphoreType.DMA,
        ],
        compiler_params=pltpu.CompilerParams(use_tc_tiling_on_sc=False),
    )
    def k(dst_hbm, ones_hbm, z_hbm, out0, out1, dst_v, ones_v, acc_sh, sem):
        c = lax.axis_index("c")
        s = lax.axis_index("s")
        wid = c * _NS + s
        base = s * rpt
        pltpu.sync_copy(z_hbm.at[pl.ds(base, rpt)], acc_sh.at[pl.ds(base, rpt)])
        pltpu.sync_copy(dst_hbm.at[wid], dst_v)
        pltpu.sync_copy(ones_hbm, ones_v)
        plsc.subcore_barrier()

        def body(j, carry):
            pltpu.async_copy(ones_v, acc_sh.at[dst_v.at[j]], sem, add=True)
            return carry

        lax.fori_loop(0, nch, body, 0)

        def drain(j, carry):
            pltpu.make_async_copy(ones_v, acc_sh.at[dst_v.at[0]], sem).wait()
            return carry

        lax.fori_loop(0, nch, drain, 0)
        plsc.subcore_barrier()

        @pl.when(c == 0)
        def _():
            pltpu.sync_copy(acc_sh.at[pl.ds(base, rpt)], out0.at[pl.ds(base, rpt)])

        @pl.when(c == 1)
        def _():
            pltpu.sync_copy(acc_sh.at[pl.ds(base, rpt)], out1.at[pl.ds(base, rpt)])

    return k(dst3, ones_l, zeros_n)


def _scatter_rows(src3, dst3, g, zeros_rows, np_, nch):
    rpt = np_ // _NS

    @functools.partial(
        pl.kernel,
        out_type=[jax.ShapeDtypeStruct((np_, _HP), jnp.float32),
                  jax.ShapeDtypeStruct((np_, _HP), jnp.float32)],
        mesh=_sc_mesh(),
        scratch_types=[
            pltpu.VMEM((nch, _CHUNK), jnp.int32),
            pltpu.VMEM((nch, _CHUNK), jnp.int32),
            pltpu.VMEM((_NBUF, _CHUNK, _HP), jnp.float32),
            pltpu.VMEM_SHARED((np_, _HP), jnp.float32),
            pltpu.SemaphoreType.DMA((_NBUF,)),
            pltpu.SemaphoreType.DMA((_NBUF,)),
        ],
        compiler_params=pltpu.CompilerParams(use_tc_tiling_on_sc=False),
    )
    def k(src_hbm, dst_hbm, g_hbm, z_hbm, out0, out1,
          src_v, dst_v, rows_v, acc_sh, gsem, ssem):
        c = lax.axis_index("c")
        s = lax.axis_index("s")
        wid = c * _NS + s
        base = s * rpt

        @pl.when(c == 0)
        def _():
            pltpu.sync_copy(g_hbm.at[pl.ds(base, rpt)], acc_sh.at[pl.ds(base, rpt)])

        @pl.when(c == 1)
        def _():
            pltpu.sync_copy(z_hbm.at[pl.ds(base, rpt)], acc_sh.at[pl.ds(base, rpt)])

        pltpu.sync_copy(src_hbm.at[wid], src_v)
        pltpu.sync_copy(dst_hbm.at[wid], dst_v)
        plsc.subcore_barrier()

        def body(j, carry):
            b = lax.rem(j, _NBUF)

            @pl.when(j < nch)
            def _():
                @pl.when(j >= _NBUF)
                def _():
                    pltpu.make_async_copy(
                        rows_v.at[b], acc_sh.at[dst_v.at[j - _NBUF]],
                        ssem.at[b]).wait()
                pltpu.async_copy(g_hbm.at[src_v.at[j]], rows_v.at[b],
                                 gsem.at[b])

            i = j - _LAG

            @pl.when((i >= 0) & (i < nch))
            def _():
                bi = lax.rem(i + _NBUF, _NBUF)
                pltpu.make_async_copy(g_hbm.at[src_v.at[i]], rows_v.at[bi],
                                      gsem.at[bi]).wait()
                pltpu.async_copy(rows_v.at[bi], acc_sh.at[dst_v.at[i]],
                                 ssem.at[bi], add=True)

            return carry

        lax.fori_loop(0, nch + _LAG, body, 0)
        for b in range(_NBUF):
            pltpu.make_async_copy(rows_v.at[b], acc_sh.at[dst_v.at[0]],
                                  ssem.at[b]).wait()
        plsc.subcore_barrier()

        @pl.when(c == 0)
        def _():
            pltpu.sync_copy(acc_sh.at[pl.ds(base, rpt)], out0.at[pl.ds(base, rpt)])

        @pl.when(c == 1)
        def _():
            pltpu.sync_copy(acc_sh.at[pl.ds(base, rpt)], out1.at[pl.ds(base, rpt)])

    return k(src3, dst3, g, zeros_rows)


def _tc_matmul0(xt, w0t):
    np_ = xt.shape[1]
    hp = w0t.shape[0]

    def body(x_ref, w_ref, h_ref):
        h_ref[...] = jnp.dot(w_ref[...], x_ref[...],
                             preferred_element_type=jnp.float32)

    return pl.pallas_call(
        body,
        out_shape=jax.ShapeDtypeStruct((hp, np_), jnp.float32),
    )(xt, w0t)


def _tc_prep(h0t, d0, d1):
    def body(h_ref, d0_ref, d1_ref, g_ref, dinv_ref):
        dinv = lax.rsqrt(d0_ref[...] + d1_ref[...] + 1.0)
        g_ref[...] = h_ref[...] * dinv
        dinv_ref[...] = dinv

    return pl.pallas_call(
        body,
        out_shape=[
            jax.ShapeDtypeStruct(h0t.shape, jnp.float32),
            jax.ShapeDtypeStruct(d0.shape, jnp.float32),
        ],
    )(h0t, d0, d1)


def _tc_mid(p0t, p1t, dinvt, bt, wt):
    def body(p0_ref, p1_ref, dinv_ref, b_ref, w_ref, g_ref):
        dinv = dinv_ref[...]
        h = jnp.maximum((p0_ref[...] + p1_ref[...]) * dinv + b_ref[...], 0.0)
        g_ref[...] = jnp.dot(w_ref[...], h,
                             preferred_element_type=jnp.float32) * dinv

    return pl.pallas_call(
        body,
        out_shape=jax.ShapeDtypeStruct(p0t.shape, jnp.float32),
    )(p0t, p1t, dinvt, bt, wt)


def _tc_final(p0t, p1t, dinvt, b3t, wlt, blt):
    np_ = p0t.shape[1]
    n_cls = wlt.shape[0]

    def body(p0_ref, p1_ref, dinv_ref, b_ref, w_ref, bl_ref, o_ref):
        h = (p0_ref[...] + p1_ref[...]) * dinv_ref[...] + b_ref[...]
        o_ref[...] = jnp.dot(w_ref[...], h,
                             preferred_element_type=jnp.float32) + bl_ref[...]

    return pl.pallas_call(
        body,
        out_shape=jax.ShapeDtypeStruct((n_cls, np_), jnp.float32),
    )(p0t, p1t, dinvt, b3t, wlt, blt)


def kernel(x, edge_index, batch, W0, b0, W1, b1, W2, b2, W3, b3, Wl, bl):
    n, f_in = x.shape
    e = edge_index.shape[1]
    h = W0.shape[1]

    np_ = ((n + 2047) // 2048) * 2048
    n_pad_rows = np_ - n

    slab = _NW * _CHUNK
    ep = ((e + slab - 1) // slab) * slab
    nch = ep // slab
    pad = ep - e
    pad_idx = n + (jnp.arange(pad, dtype=jnp.int32) % n_pad_rows)
    src3 = jnp.concatenate([edge_index[0], pad_idx]).reshape(_NW, nch, _CHUNK)
    dst3 = jnp.concatenate([edge_index[1], pad_idx]).reshape(_NW, nch, _CHUNK)

    zeros_rows = jnp.zeros((np_, _HP), jnp.float32)
    zeros_n = jnp.zeros((np_,), jnp.float32)
    ones_l = jnp.ones((_CHUNK,), jnp.float32)

    xt = jnp.pad(x, ((0, np_ - n), (0, 0))).T
    w0t = jnp.pad(W0, ((0, 0), (0, _HP - h))).T
    w1t = jnp.pad(W1, ((0, _HP - h), (0, _HP - h))).T
    w2t = jnp.pad(W2, ((0, _HP - h), (0, _HP - h))).T
    w3t = jnp.pad(W3, ((0, _HP - h), (0, _HP - h))).T
    wlt = jnp.pad(Wl, ((0, _HP - h), (0, 0))).T
    b0t = jnp.pad(b0, (0, _HP - h))[:, None]
    b1t = jnp.pad(b1, (0, _HP - h))[:, None]
    b2t = jnp.pad(b2, (0, _HP - h))[:, None]
    b3t = jnp.pad(b3, (0, _HP - h))[:, None]
    blt = bl[:, None]

    h0t = _tc_matmul0(xt, w0t)
    d0, d1 = _degrees(dst3, ones_l, zeros_n, np_, nch)
    gt, dinvt = _tc_prep(h0t, d0[None, :], d1[None, :])

    for bt, wt in ((b0t, w1t), (b1t, w2t), (b2t, w3t)):
        p0, p1 = _scatter_rows(src3, dst3, gt.T, zeros_rows, np_, nch)
        gt = _tc_mid(p0.T, p1.T, dinvt, bt, wt)

    p0, p1 = _scatter_rows(src3, dst3, gt.T, zeros_rows, np_, nch)
    out = _tc_final(p0.T, p1.T, dinvt, b3t, wlt, blt)
    return out.T[:n]
